# baseline (speedup 1.0000x reference)
import functools

import jax
import jax.numpy as jnp
from jax.experimental import pallas as pl
from jax.experimental.pallas import tpu as pltpu

_B, _S, _D = 4, 2048, 768
_M, _K, _R, _RAD = 32, 8, 64, 8.0

_HP = jax.lax.Precision.HIGHEST


def _dot(a, b, dims, precision=_HP):
    return jax.lax.dot_general(a, b, (dims, ((), ())),
                               precision=precision,
                               preferred_element_type=jnp.float32)


def _body1(pt_ref, w_ref, b_ref, modes_ref, idx_ref, tks_ref, maskf_ref,
           spread_ref, denoms_ref, anch_ref, wts_ref, ent_ref):
    b = pl.program_id(0)
    pt = pt_ref[0]

    sc = _dot(w_ref[...], pt, ((1,), (1,))) + b_ref[0, 0]

    lane_s = jax.lax.broadcasted_iota(jnp.int32, (1, _S), 1)
    k_lane = jax.lax.broadcasted_iota(jnp.int32, (1, _K), 1)
    k_sub = jax.lax.broadcasted_iota(jnp.int32, (_K, 1), 0)

    cur = sc
    idx_row = jnp.zeros((1, _K), jnp.int32)
    val_row = jnp.zeros((1, _K), jnp.float32)
    idxf_col = jnp.zeros((_K, 1), jnp.float32)
    rows = []
    for k in range(_K):
        m = jnp.max(cur)
        i = jnp.min(jnp.where(cur == m, lane_s, _S))
        idx_row = jnp.where(k_lane == k, i, idx_row)
        val_row = jnp.where(k_lane == k, m, val_row)
        idxf_col = jnp.where(k_sub == k, i.astype(jnp.float32), idxf_col)
        blk = pt_ref[0, pl.ds(pl.multiple_of((i // 8) * 8, 8), 8), :]
        sel = (jax.lax.broadcasted_iota(jnp.int32, (8, 1), 0) == i % 8)
        rows.append(jnp.sum(blk * sel.astype(jnp.float32), axis=0,
                            keepdims=True))
        cur = jnp.where(lane_s == i, -jnp.inf, cur)
    anchors = jnp.concatenate(rows, axis=0)

    idx_ref[...] = idx_row.reshape(1, 1, _K)
    tks_ref[...] = val_row.reshape(1, 1, _K)
    anch_ref[...] = anchors.reshape(1, _K, _D)

    an = jnp.sqrt(jnp.sum(anchors * anchors, axis=1, keepdims=True))
    anorm = anchors / jnp.maximum(an, 1e-6)
    modes = modes_ref[...]
    mn = jnp.sqrt(jnp.sum(modes * modes, axis=1, keepdims=True))
    mnorm = modes / jnp.maximum(mn, 1e-6)
    logits = _dot(anorm, mnorm, ((1,), (1,)))
    lmax = jnp.max(logits, axis=1, keepdims=True)
    ex = jnp.exp(logits - lmax)
    wts = ex / jnp.sum(ex, axis=1, keepdims=True)
    wts_ref[...] = wts.reshape(1, _K, _M)
    ent_b = -jnp.sum(wts * jnp.log(jnp.maximum(wts, 1e-8)))

    pos = jax.lax.broadcasted_iota(jnp.int32, (_K, _S), 1).astype(jnp.float32)
    dist = jnp.abs(pos - idxf_col)
    nb = (dist <= _RAD).astype(jnp.float32)
    ew = jnp.exp(-dist / _RAD) * nb
    spread_ref[0] = ew / jnp.sum(ew, axis=1, keepdims=True)
    denoms_ref[...] = _dot(jnp.ones((1, _S), jnp.float32), ew,
                           ((1,), (1,))).reshape(1, 1, _K)

    onehot = (dist == 0.0).astype(jnp.float32)
    maskf_ref[...] = jnp.max(onehot, axis=0, keepdims=True).reshape(1, 1, _S)

    prev = jnp.where(b == 0, jnp.zeros((1, 1), jnp.float32), ent_ref[...])
    tot = prev + ent_b
    ent_ref[...] = jnp.where(b == _B - 1, tot / (_B * _K), tot)


def _body2(idx_sref, ts_ref, anchf_ref, wtsf_ref, a_ref, bt_ref, denoms_ref,
           g_ref, be_ref, aug_ref, lvar_ref, trans_ref):
    b = pl.program_id(0)

    @pl.when(b == 0)
    def _translate():
        mr_sub = jax.lax.broadcasted_iota(jnp.int32, (_M, _M * _R), 0)
        mr_lane = jax.lax.broadcasted_iota(jnp.int32, (_M, _M * _R), 1)
        expand = (mr_lane // _R == mr_sub).astype(jnp.float32)
        r_sub = jax.lax.broadcasted_iota(jnp.int32, (_M * _R, _R), 0)
        r_lane = jax.lax.broadcasted_iota(jnp.int32, (_M * _R, _R), 1)
        fold = (r_sub % _R == r_lane).astype(jnp.float32)

        anc16 = anchf_ref[...].astype(jnp.bfloat16)
        q = _dot(anc16, a_ref[...], ((1,), (0,)), precision=None)
        wrep = _dot(wtsf_ref[...], expand, ((1,), (0,)))
        low = _dot(q * wrep, fold, ((1,), (0,)))
        lowrep = _dot(low, fold, ((1,), (1,)))
        lw = (wrep * lowrep).astype(jnp.bfloat16)
        trans_ref[...] = _dot(lw, bt_ref[...], ((1,), (0,)), precision=None)

    def _ln(xx):
        mu = jnp.mean(xx, axis=1, keepdims=True)
        xc = xx - mu
        v = jnp.mean(xc * xc, axis=1, keepdims=True)
        return xc / jnp.sqrt(v + 1e-5) * g_ref[...] + be_ref[...]

    ts = ts_ref[0]
    aug_ref[0] = _ln(ts)

    trans_b = trans_ref[pl.ds(pl.multiple_of(b * _K, 8), _K), :]
    denom_row = denoms_ref[0]
    k_lane = jax.lax.broadcasted_iota(jnp.int32, (1, _K), 1)
    win_sub = jax.lax.broadcasted_iota(jnp.int32, (32, 1), 0)

    idxf_row = jnp.zeros((1, _K), jnp.float32)
    idx_scalars = []
    for k in range(_K):
        i = idx_sref[b, k]
        idx_scalars.append(i)
        idxf_row = jnp.where(k_lane == k, i.astype(jnp.float32), idxf_row)

    lvar_row = jnp.zeros((1, _K), jnp.float32)
    for k in range(_K):
        i = idx_scalars[k]
        st = pl.multiple_of(jnp.clip(((i - 8) // 8) * 8, 0, _S - 32), 8)
        ws = ts_ref[0, pl.ds(st, 32), :]
        posw = win_sub + st
        mcol = (jnp.abs(posw - i) <= 8).astype(jnp.float32)
        cnt = jnp.maximum(jnp.sum(mcol), 1.0)
        wmean = jnp.sum(ws * mcol, axis=0, keepdims=True) / cnt
        wex2 = jnp.sum(ws * ws * mcol, axis=0, keepdims=True) / cnt
        lv = jnp.sum(wex2 - wmean * wmean) / _D
        lvar_row = jnp.where(k_lane == k, lv, lvar_row)

        dw = jnp.abs(posw.astype(jnp.float32) - idxf_row)
        eww = jnp.where(dw <= _RAD, jnp.exp(-dw / _RAD), 0.0)
        sw = eww / denom_row
        updw = _dot(sw, trans_b, ((1,), (0,)))
        aug_ref[0, pl.ds(st, 32), :] = _ln(ws + updw)

    lvar_ref[...] = lvar_row.reshape(1, 1, _K)


@functools.partial(jax.jit, static_argnames=())
def kernel(pt_hidden, ts_hidden, patch_positions, mode_centroids, w_score,
           b_score, translation_a, translation_b, ln_gamma, ln_beta):
    del patch_positions
    aflat = jnp.transpose(translation_a, (1, 0, 2)).reshape(
        _D, _M * _R).astype(jnp.bfloat16)
    bflat = jnp.transpose(translation_b, (0, 2, 1)).reshape(
        _M * _R, _D).astype(jnp.bfloat16)
    b2 = b_score.reshape(1, 1)
    g2 = ln_gamma.reshape(1, _D)
    be2 = ln_beta.reshape(1, _D)

    const = lambda shape: pl.BlockSpec(shape, lambda b: (0,) * len(shape))
    batched = lambda shape: pl.BlockSpec(shape,
                                         lambda b: (b,) + (0,) * (len(shape) - 1))

    idx3, tks3, maskf3, spread, denoms3, anch, wtsb, ent = pl.pallas_call(
        _body1,
        grid=(_B,),
        in_specs=[
            batched((1, _S, _D)),
            const((1, _D)),
            const((1, 1)),
            const((_M, _D)),
        ],
        out_specs=[
            batched((1, 1, _K)),
            batched((1, 1, _K)),
            batched((1, 1, _S)),
            batched((1, _K, _S)),
            batched((1, 1, _K)),
            batched((1, _K, _D)),
            batched((1, _K, _M)),
            const((1, 1)),
        ],
        out_shape=[
            jax.ShapeDtypeStruct((_B, 1, _K), jnp.int32),
            jax.ShapeDtypeStruct((_B, 1, _K), jnp.float32),
            jax.ShapeDtypeStruct((_B, 1, _S), jnp.float32),
            jax.ShapeDtypeStruct((_B, _K, _S), jnp.float32),
            jax.ShapeDtypeStruct((_B, 1, _K), jnp.float32),
            jax.ShapeDtypeStruct((_B, _K, _D), jnp.float32),
            jax.ShapeDtypeStruct((_B, _K, _M), jnp.float32),
            jax.ShapeDtypeStruct((1, 1), jnp.float32),
        ],
    )(pt_hidden, w_score, b2, mode_centroids)

    anchor_idx = idx3.reshape(_B, _K)

    cb = lambda shape: pl.BlockSpec(shape, lambda b, sref: (0,) * len(shape))
    bb = lambda shape: pl.BlockSpec(
        shape, lambda b, sref: (b,) + (0,) * (len(shape) - 1))
    grid_spec = pltpu.PrefetchScalarGridSpec(
        num_scalar_prefetch=1,
        grid=(_B,),
        in_specs=[
            bb((1, _S, _D)),
            cb((_B * _K, _D)),
            cb((_B * _K, _M)),
            cb((_D, _M * _R)),
            cb((_M * _R, _D)),
            bb((1, 1, _K)),
            cb((1, _D)),
            cb((1, _D)),
        ],
        out_specs=[
            bb((1, _S, _D)),
            bb((1, 1, _K)),
        ],
        scratch_shapes=[pltpu.VMEM((_B * _K, _D), jnp.float32)],
    )
    aug, lvar3 = pl.pallas_call(
        _body2,
        grid_spec=grid_spec,
        out_shape=[
            jax.ShapeDtypeStruct((_B, _S, _D), jnp.float32),
            jax.ShapeDtypeStruct((_B, 1, _K), jnp.float32),
        ],
    )(anchor_idx, ts_hidden, anch.reshape(_B * _K, _D),
      wtsb.reshape(_B * _K, _M), aflat, bflat, denoms3, g2, be2)

    topk_scores = tks3.reshape(_B, _K)
    anchor_mask = maskf3.reshape(_B, _S).astype(bool)
    local_variance = lvar3.reshape(_B, _K, 1)
    injection_gate = jnp.ones((_B, _K, 1), jnp.float32)
    mode_entropy = ent[0, 0]
    return (aug, anchor_idx, anchor_mask, topk_scores, injection_gate,
            local_variance, spread, mode_entropy)


# speedup vs baseline: 1.0481x; 1.0481x over previous
import functools

import jax
import jax.numpy as jnp
from jax.experimental import pallas as pl
from jax.experimental.pallas import tpu as pltpu

_B, _S, _D = 4, 2048, 768
_M, _K, _R, _RAD = 32, 8, 64, 8.0

_HI = jax.lax.Precision.HIGHEST


def _dot(a, b, dims, precision=_HI):
    return jax.lax.dot_general(a, b, (dims, ((), ())),
                               precision=precision,
                               preferred_element_type=jnp.float32)


def _body(pt_ref, ts_ref, w_ref, b_ref, modes_ref, a_ref, bt_ref, g_ref,
          be_ref, aug_ref, idx_ref, tks_ref, maskf_ref, spread_ref, lvar_ref,
          ent_ref, anch_s, wts_s, trans_s, idx_s):
    s = pl.program_id(0)

    @pl.when(s < _B)
    def _phase1():
        b = s
        pt = pt_ref[0]
        sc = _dot(w_ref[...], pt, ((1,), (1,))) + b_ref[0, 0]

        lane_s = jax.lax.broadcasted_iota(jnp.int32, (1, _S), 1)
        k_lane = jax.lax.broadcasted_iota(jnp.int32, (1, _K), 1)
        k_sub = jax.lax.broadcasted_iota(jnp.int32, (_K, 1), 0)

        cur = sc
        idx_row = jnp.zeros((1, _K), jnp.int32)
        val_row = jnp.zeros((1, _K), jnp.float32)
        idxf_col = jnp.zeros((_K, 1), jnp.float32)
        rows = []
        for k in range(_K):
            m = jnp.max(cur)
            i = jnp.min(jnp.where(cur == m, lane_s, _S))
            idx_row = jnp.where(k_lane == k, i, idx_row)
            val_row = jnp.where(k_lane == k, m, val_row)
            idxf_col = jnp.where(k_sub == k, i.astype(jnp.float32), idxf_col)
            idx_s[b, k] = i
            blk = pt_ref[0, pl.ds(pl.multiple_of((i // 8) * 8, 8), 8), :]
            sel = (jax.lax.broadcasted_iota(jnp.int32, (8, 1), 0) == i % 8)
            rows.append(jnp.sum(blk * sel.astype(jnp.float32), axis=0,
                                keepdims=True))
            cur = jnp.where(lane_s == i, -jnp.inf, cur)
        anchors = jnp.concatenate(rows, axis=0)

        idx_ref[...] = idx_row.reshape(1, 1, _K)
        tks_ref[...] = val_row.reshape(1, 1, _K)
        anch_s[pl.ds(pl.multiple_of(b * _K, 8), _K), :] = anchors

        an = jnp.sqrt(jnp.sum(anchors * anchors, axis=1, keepdims=True))
        anorm = anchors / jnp.maximum(an, 1e-6)
        modes = modes_ref[...]
        mn = jnp.sqrt(jnp.sum(modes * modes, axis=1, keepdims=True))
        mnorm = modes / jnp.maximum(mn, 1e-6)
        logits = _dot(anorm, mnorm, ((1,), (1,)))
        lmax = jnp.max(logits, axis=1, keepdims=True)
        ex = jnp.exp(logits - lmax)
        wts = ex / jnp.sum(ex, axis=1, keepdims=True)
        wts_s[pl.ds(pl.multiple_of(b * _K, 8), _K), :] = wts
        ent_b = -jnp.sum(wts * jnp.log(jnp.maximum(wts, 1e-8)))

        pos = jax.lax.broadcasted_iota(jnp.int32, (_K, _S),
                                       1).astype(jnp.float32)
        dist = jnp.abs(pos - idxf_col)
        nb = (dist <= _RAD).astype(jnp.float32)
        ew = jnp.exp(-dist / _RAD) * nb
        spread_ref[0] = ew / jnp.sum(ew, axis=1, keepdims=True)

        onehot = (dist == 0.0).astype(jnp.float32)
        maskf_ref[...] = jnp.max(onehot, axis=0,
                                 keepdims=True).reshape(1, 1, _S)

        prev = jnp.where(b == 0, jnp.zeros((1, 1), jnp.float32),
                         ent_ref[...])
        tot = prev + ent_b
        ent_ref[...] = jnp.where(b == _B - 1, tot / (_B * _K), tot)

    @pl.when(s == _B)
    def _translate():
        mr_sub = jax.lax.broadcasted_iota(jnp.int32, (_M, _M * _R), 0)
        mr_lane = jax.lax.broadcasted_iota(jnp.int32, (_M, _M * _R), 1)
        expand = (mr_lane // _R == mr_sub).astype(jnp.float32)
        r_sub = jax.lax.broadcasted_iota(jnp.int32, (_M * _R, _R), 0)
        r_lane = jax.lax.broadcasted_iota(jnp.int32, (_M * _R, _R), 1)
        fold = (r_sub % _R == r_lane).astype(jnp.float32)

        anc16 = anch_s[...].astype(jnp.bfloat16)
        q = _dot(anc16, a_ref[...], ((1,), (0,)), precision=None)
        wrep = _dot(wts_s[...], expand, ((1,), (0,)))
        low = _dot(q * wrep, fold, ((1,), (0,)))
        lowrep = _dot(low, fold, ((1,), (1,)))
        lw = (wrep * lowrep).astype(jnp.bfloat16)
        trans_s[...] = _dot(lw, bt_ref[...], ((1,), (0,)), precision=None)

    @pl.when(s >= _B)
    def _phase2():
        b = s - _B

        def _ln(xx):
            mu = jnp.mean(xx, axis=1, keepdims=True)
            xc = xx - mu
            v = jnp.mean(xc * xc, axis=1, keepdims=True)
            return xc / jnp.sqrt(v + 1e-5) * g_ref[...] + be_ref[...]

        aug_ref[0] = _ln(ts_ref[0])

        trans_b = trans_s[pl.ds(pl.multiple_of(b * _K, 8), _K), :]
        k_lane = jax.lax.broadcasted_iota(jnp.int32, (1, _K), 1)
        win_sub = jax.lax.broadcasted_iota(jnp.int32, (32, 1), 0)
        t_lane = jax.lax.broadcasted_iota(jnp.int32, (1, 32), 1) - 8

        idxf_row = jnp.zeros((1, _K), jnp.float32)
        denom_row = jnp.zeros((1, _K), jnp.float32)
        idx_scalars = []
        for k in range(_K):
            i = idx_s[b, k]
            idx_scalars.append(i)
            idxf_row = jnp.where(k_lane == k, i.astype(jnp.float32),
                                 idxf_row)
            valid = ((jnp.abs(t_lane) <= 8) & (t_lane + i >= 0)
                     & (t_lane + i < _S))
            et = jnp.where(valid,
                           jnp.exp(-jnp.abs(t_lane).astype(jnp.float32)
                                   / _RAD), 0.0)
            denom_row = jnp.where(k_lane == k, jnp.sum(et), denom_row)

        lvar_row = jnp.zeros((1, _K), jnp.float32)
        for k in range(_K):
            i = idx_scalars[k]
            st = pl.multiple_of(jnp.clip(((i - 8) // 8) * 8, 0, _S - 32), 8)
            ws = ts_ref[0, pl.ds(st, 32), :]
            posw = win_sub + st
            mcol = (jnp.abs(posw - i) <= 8).astype(jnp.float32)
            cnt = jnp.maximum(jnp.sum(mcol), 1.0)
            wmean = jnp.sum(ws * mcol, axis=0, keepdims=True) / cnt
            wex2 = jnp.sum(ws * ws * mcol, axis=0, keepdims=True) / cnt
            lv = jnp.sum(wex2 - wmean * wmean) / _D
            lvar_row = jnp.where(k_lane == k, lv, lvar_row)

            dw = jnp.abs(posw.astype(jnp.float32) - idxf_row)
            eww = jnp.where(dw <= _RAD, jnp.exp(-dw / _RAD), 0.0)
            sw = eww / denom_row
            updw = _dot(sw, trans_b, ((1,), (0,)))
            aug_ref[0, pl.ds(st, 32), :] = _ln(ws + updw)

        lvar_ref[...] = lvar_row.reshape(1, 1, _K)


@functools.partial(jax.jit, static_argnames=())
def kernel(pt_hidden, ts_hidden, patch_positions, mode_centroids, w_score,
           b_score, translation_a, translation_b, ln_gamma, ln_beta):
    del patch_positions
    aflat = jnp.transpose(translation_a, (1, 0, 2)).reshape(
        _D, _M * _R).astype(jnp.bfloat16)
    bflat = jnp.transpose(translation_b, (0, 2, 1)).reshape(
        _M * _R, _D).astype(jnp.bfloat16)
    b2 = b_score.reshape(1, 1)
    g2 = ln_gamma.reshape(1, _D)
    be2 = ln_beta.reshape(1, _D)

    const = lambda shape: pl.BlockSpec(shape, lambda s: (0,) * len(shape))

    def p1(shape):
        return pl.BlockSpec(
            shape, lambda s: (jnp.minimum(s, _B - 1),) + (0,) * (len(shape) - 1))

    def p2(shape):
        return pl.BlockSpec(
            shape, lambda s: (jnp.maximum(s - _B, 0),) + (0,) * (len(shape) - 1))

    aug, idx3, tks3, maskf3, spread, lvar3, ent = pl.pallas_call(
        _body,
        grid=(2 * _B,),
        in_specs=[
            p1((1, _S, _D)),
            p2((1, _S, _D)),
            const((1, _D)),
            const((1, 1)),
            const((_M, _D)),
            const((_D, _M * _R)),
            const((_M * _R, _D)),
            const((1, _D)),
            const((1, _D)),
        ],
        out_specs=[
            p2((1, _S, _D)),
            p1((1, 1, _K)),
            p1((1, 1, _K)),
            p1((1, 1, _S)),
            p1((1, _K, _S)),
            p2((1, 1, _K)),
            const((1, 1)),
        ],
        out_shape=[
            jax.ShapeDtypeStruct((_B, _S, _D), jnp.float32),
            jax.ShapeDtypeStruct((_B, 1, _K), jnp.int32),
            jax.ShapeDtypeStruct((_B, 1, _K), jnp.float32),
            jax.ShapeDtypeStruct((_B, 1, _S), jnp.float32),
            jax.ShapeDtypeStruct((_B, _K, _S), jnp.float32),
            jax.ShapeDtypeStruct((_B, 1, _K), jnp.float32),
            jax.ShapeDtypeStruct((1, 1), jnp.float32),
        ],
        scratch_shapes=[
            pltpu.VMEM((_B * _K, _D), jnp.float32),
            pltpu.VMEM((_B * _K, _M), jnp.float32),
            pltpu.VMEM((_B * _K, _D), jnp.float32),
            pltpu.SMEM((_B, _K), jnp.int32),
        ],
    )(pt_hidden, ts_hidden, w_score, b2, mode_centroids, aflat, bflat, g2,
      be2)

    anchor_idx = idx3.reshape(_B, _K)
    topk_scores = tks3.reshape(_B, _K)
    anchor_mask = maskf3.reshape(_B, _S).astype(bool)
    local_variance = lvar3.reshape(_B, _K, 1)
    injection_gate = jnp.ones((_B, _K, 1), jnp.float32)
    mode_entropy = ent[0, 0]
    return (aug, anchor_idx, anchor_mask, topk_scores, injection_gate,
            local_variance, spread, mode_entropy)
